# Initial kernel scaffold; baseline (speedup 1.0000x reference)
import jax
import jax.numpy as jnp
from jax.experimental import pallas as pl


def kernel(x, edge_index, W_in1, b_in1, W_in2, b_in2, W_msg, b_msg, W_c1, b_c1, W_c2, b_c2, ln_g, ln_b, W_out, b_out):
    raise NotImplementedError("write your pallas kernel here")


# speedup vs baseline: 6.8157x; 6.8157x over previous
import functools

import jax
import jax.numpy as jnp
from jax import lax
from jax.experimental import pallas as pl
from jax.experimental.pallas import tpu as pltpu
from jax.experimental.pallas import tpu_sc as plsc

N = 10000
E = 320000
D = 128
C = 10
NUM_STEPS = 4

NC = 2
NS = 16
NW = NC * NS
EPT = E // NW
CK = 80
CH = EPT // CK
NP = 10240
RPT = NP // NS

ROWS_TC = 1000


def _sc_agg(p, src3, dst3, zblk):
    mesh = plsc.VectorSubcoreMesh(core_axis_name="c", subcore_axis_name="s")

    @functools.partial(
        pl.kernel,
        out_type=jax.ShapeDtypeStruct((NC, NP, D), jnp.float32),
        mesh=mesh,
        scratch_types=[
            pltpu.VMEM((CH, CK), jnp.int32),
            pltpu.VMEM((CH, CK), jnp.int32),
            pltpu.VMEM((CK, D), jnp.float32),
            pltpu.VMEM_SHARED((NP, D), jnp.float32),
            pltpu.SemaphoreType.DMA,
        ],
    )
    def k(p_hbm, src_hbm, dst_hbm, z_hbm, out_hbm, src_v, dst_v, rows_v,
          acc_sh, sem):
        c = lax.axis_index("c")
        s = lax.axis_index("s")
        wid = c * NS + s
        pltpu.sync_copy(src_hbm.at[wid], src_v)
        pltpu.sync_copy(dst_hbm.at[wid], dst_v)
        pltpu.sync_copy(z_hbm, acc_sh.at[pl.ds(s * RPT, RPT)])
        plsc.subcore_barrier()

        def chunk(i, carry):
            pltpu.async_copy(p_hbm.at[src_v.at[i]], rows_v, sem).wait()
            pltpu.sync_copy(rows_v, acc_sh.at[dst_v.at[i]], add=True)
            return carry

        lax.fori_loop(0, CH, chunk, 0)
        plsc.subcore_barrier()
        pltpu.sync_copy(acc_sh.at[pl.ds(s * RPT, RPT)],
                        out_hbm.at[c, pl.ds(s * RPT, RPT)])

    return k(p, src3, dst3, zblk)


def _tc_in(x, W1, b1, W2, b2, Wm, bm):
    def body(x_ref, w1, bb1, w2, bb2, wm, bbm, h_ref, p_ref):
        t = jnp.maximum(x_ref[...] @ w1[...] + bb1[...], 0.0)
        h = t @ w2[...] + bb2[...]
        h_ref[...] = h
        p_ref[...] = jnp.maximum(h @ wm[...] + bbm[...], 0.0)

    full = lambda shape: pl.BlockSpec(shape, lambda i: (0, 0))
    rows = pl.BlockSpec((ROWS_TC, D), lambda i: (i, 0))
    return pl.pallas_call(
        body,
        grid=(N // ROWS_TC,),
        in_specs=[rows, full((D, 2 * D)), full((1, 2 * D)), full((2 * D, D)),
                  full((1, D)), full((D, D)), full((1, D))],
        out_specs=[rows, rows],
        out_shape=[jax.ShapeDtypeStruct((N, D), jnp.float32),
                   jax.ShapeDtypeStruct((N, D), jnp.float32)],
    )(x, W1, b1, W2, b2, Wm, bm)


def _tc_upd(h, aggs, W1h, W1a, b1, W2, b2, g, b, Wp, bp, last):
    def body(h_ref, a0, a1, w1h, w1a, bb1, w2, bb2, gg, bb, wp, bbp,
             hn_ref, p_ref):
        h_blk = h_ref[...]
        agg = a0[...] + a1[...]
        t = jnp.maximum(h_blk @ w1h[...] + agg @ w1a[...] + bb1[...], 0.0)
        z = h_blk + t @ w2[...] + bb2[...]
        mu = jnp.mean(z, axis=-1, keepdims=True)
        zc = z - mu
        var = jnp.mean(zc * zc, axis=-1, keepdims=True)
        hn = zc * lax.rsqrt(var + 1e-5) * gg[...] + bb[...]
        hn_ref[...] = hn
        p_ref[...] = (hn @ wp[...] + bbp[...] if last
                      else jnp.maximum(hn @ wp[...] + bbp[...], 0.0))

    pdim = C if last else D
    full = lambda shape: pl.BlockSpec(shape, lambda i: (0, 0))
    rows = pl.BlockSpec((ROWS_TC, D), lambda i: (i, 0))
    prows = pl.BlockSpec((ROWS_TC, pdim), lambda i: (i, 0))
    return pl.pallas_call(
        body,
        grid=(N // ROWS_TC,),
        in_specs=[rows, rows, rows, full((D, 4 * D)), full((D, 4 * D)),
                  full((1, 4 * D)), full((4 * D, D)), full((1, D)),
                  full((1, D)), full((1, D)), full((D, pdim)),
                  full((1, pdim))],
        out_specs=[rows, prows],
        out_shape=[jax.ShapeDtypeStruct((N, D), jnp.float32),
                   jax.ShapeDtypeStruct((N, pdim), jnp.float32)],
    )(h, aggs[0], aggs[1], W1h, W1a, b1, W2, b2, g, b, Wp, bp)


def kernel(x, edge_index, W_in1, b_in1, W_in2, b_in2, W_msg, b_msg,
           W_c1, b_c1, W_c2, b_c2, ln_g, ln_b, W_out, b_out):
    src3 = edge_index[0].reshape(NW, CH, CK)
    dst3 = edge_index[1].reshape(NW, CH, CK)
    zblk = jnp.zeros((RPT, D), jnp.float32)

    r1 = lambda v: v.reshape(1, -1)
    h, p = _tc_in(x, W_in1, r1(b_in1), W_in2, r1(b_in2), W_msg, r1(b_msg))
    W1h = jax.lax.slice_in_dim(W_c1, 0, D, axis=0)
    W1a = jax.lax.slice_in_dim(W_c1, D, 2 * D, axis=0)
    for step in range(NUM_STEPS):
        aggs = _sc_agg(p, src3, dst3, zblk)
        last = step == NUM_STEPS - 1
        Wp, bp = (W_out, b_out) if last else (W_msg, b_msg)
        h, p = _tc_upd(h, aggs, W1h, W1a, r1(b_c1), W_c2, r1(b_c2),
                       r1(ln_g), r1(ln_b), Wp, r1(bp), last)
    return p
